# baseline (speedup 1.0000x reference)
import jax
import jax.numpy as jnp
from jax.experimental import pallas as pl

EMBED_DIM = 100


def _tc_dir_broadcast(dir_col, n_rows):
    rows_blk = 4096

    def body(d_ref, o_ref):
        o_ref[...] = jnp.broadcast_to(
            d_ref[...].astype(jnp.float32), (rows_blk, EMBED_DIM)
        )

    return pl.pallas_call(
        body,
        grid=(n_rows // rows_blk,),
        in_specs=[pl.BlockSpec((rows_blk, 1), lambda i: (i, 0))],
        out_specs=pl.BlockSpec((rows_blk, EMBED_DIM), lambda i: (i, 0)),
        out_shape=jax.ShapeDtypeStruct((n_rows, EMBED_DIM), jnp.float32),
    )(dir_col)


def kernel(pkt_len_seq, pkt_dir_seq, iat_seq, pkt_len_table, iat_table):
    batch, seq = pkt_len_seq.shape
    n_idx = batch * seq

    pkt_out = jnp.take(pkt_len_table, pkt_len_seq, axis=0)
    iat_out = jnp.take(iat_table, iat_seq, axis=0)
    dir_out = _tc_dir_broadcast(pkt_dir_seq.reshape(n_idx, 1), n_idx)

    return (
        pkt_out,
        dir_out.reshape(batch, seq, EMBED_DIM),
        iat_out,
    )


# speedup vs baseline: 2.3953x; 2.3953x over previous
import jax
import jax.numpy as jnp
from jax.experimental import pallas as pl
from jax.experimental.pallas import tpu as pltpu
from jax.experimental.pallas import tpu_sc as plsc

EMBED_DIM = 100
PAD_DIM = 128
WINDOW = 128


def _sc_gather(iat_pad, pkt_pad, iat_idx, pkt_idx, n_idx):
    mesh = plsc.VectorSubcoreMesh(core_axis_name="c", subcore_axis_name="s")
    out_struct = jax.ShapeDtypeStruct((n_idx, PAD_DIM), jnp.float32)

    @pl.kernel(out_type=(out_struct, out_struct), mesh=mesh)
    def k(iat_t_hbm, pkt_t_hbm, iat_i_hbm, pkt_i_hbm, iat_o_hbm, pkt_o_hbm):
        def body(ii_vmem, pi_vmem, io_vmem, po_vmem):
            pltpu.sync_copy(iat_t_hbm.at[ii_vmem.at[0]], io_vmem)
            pltpu.sync_copy(pkt_t_hbm.at[pi_vmem.at[0]], po_vmem)

        pltpu.emit_pipeline(
            body,
            grid=(n_idx // WINDOW,),
            in_specs=[
                pl.BlockSpec((1, WINDOW), lambda i: (0, i)),
                pl.BlockSpec((1, WINDOW), lambda i: (0, i)),
            ],
            out_specs=[
                pl.BlockSpec((WINDOW, PAD_DIM), lambda i: (i, 0)),
                pl.BlockSpec((WINDOW, PAD_DIM), lambda i: (i, 0)),
            ],
            core_axis_name=("c", "s"),
            dimension_semantics=(pltpu.PARALLEL,),
        )(iat_i_hbm, pkt_i_hbm, iat_o_hbm, pkt_o_hbm)

    return k(iat_pad, pkt_pad, iat_idx, pkt_idx)


def _tc_finalize(iat_g, pkt_g, pkt_dir_seq, batch, seq):
    b_blk = 64
    out_struct = jax.ShapeDtypeStruct((batch, seq, EMBED_DIM), jnp.float32)

    def body(ig_ref, pg_ref, d_ref, io_ref, po_ref, do_ref):
        io_ref[...] = ig_ref[...][:, :EMBED_DIM].reshape(b_blk, seq, EMBED_DIM)
        po_ref[...] = pg_ref[...][:, :EMBED_DIM].reshape(b_blk, seq, EMBED_DIM)
        do_ref[...] = jnp.broadcast_to(
            d_ref[...].astype(jnp.float32)[:, :, None], (b_blk, seq, EMBED_DIM)
        )

    return pl.pallas_call(
        body,
        grid=(batch // b_blk,),
        in_specs=[
            pl.BlockSpec((b_blk * seq, PAD_DIM), lambda i: (i, 0)),
            pl.BlockSpec((b_blk * seq, PAD_DIM), lambda i: (i, 0)),
            pl.BlockSpec((b_blk, seq), lambda i: (i, 0)),
        ],
        out_specs=[
            pl.BlockSpec((b_blk, seq, EMBED_DIM), lambda i: (i, 0, 0)),
            pl.BlockSpec((b_blk, seq, EMBED_DIM), lambda i: (i, 0, 0)),
            pl.BlockSpec((b_blk, seq, EMBED_DIM), lambda i: (i, 0, 0)),
        ],
        out_shape=(out_struct, out_struct, out_struct),
    )(iat_g, pkt_g, pkt_dir_seq)


def kernel(pkt_len_seq, pkt_dir_seq, iat_seq, pkt_len_table, iat_table):
    batch, seq = pkt_len_seq.shape
    n_idx = batch * seq

    iat_pad = jnp.pad(iat_table, ((0, 0), (0, PAD_DIM - EMBED_DIM)))
    pkt_pad = jnp.pad(pkt_len_table, ((0, 0), (0, PAD_DIM - EMBED_DIM)))
    iat_idx = iat_seq.reshape(1, n_idx).astype(jnp.int32)
    pkt_idx = pkt_len_seq.reshape(1, n_idx).astype(jnp.int32)

    iat_g, pkt_g = _sc_gather(iat_pad, pkt_pad, iat_idx, pkt_idx, n_idx)
    iat_out, pkt_out, dir_out = _tc_finalize(iat_g, pkt_g, pkt_dir_seq, batch, seq)
    return (pkt_out, dir_out, iat_out)
